# baseline (speedup 1.0000x reference)
import functools

import jax
import jax.numpy as jnp
from jax import lax
from jax.experimental import pallas as pl
from jax.experimental.pallas import tpu as pltpu
from jax.experimental.pallas import tpu_sc as plsc

NC = 2
NS = 16

N_ROWS, N_COLS = 16384, 26
B = N_ROWS * N_COLS
D = 128
CHUNK = 128
IPS = N_ROWS // NS
KPJ = IPS // CHUNK
JPC = N_COLS // NC
NCHUNK = JPC * KPJ
NBLK = B // CHUNK
RBLK = N_ROWS // CHUNK
NBUF = 5
LA = NBUF - 2


def _gather_body(idx_hbm, table_hbm, out_hbm, idx_v, rows_v, gsem, ssem):
    cid = lax.axis_index("c")
    sid = lax.axis_index("s")
    pltpu.sync_copy(idx_hbm.at[:, pl.ds(sid * IPS, IPS)], idx_v)

    def src(chunk):
        jj = 2 * (chunk // KPJ) + cid
        k = chunk % KPJ
        return table_hbm.at[idx_v.at[jj, pl.ds(k * CHUNK, CHUNK)]]

    def dst(chunk):
        jj = 2 * (chunk // KPJ) + cid
        k = chunk % KPJ
        return out_hbm.at[pl.ds(jj * RBLK + sid * KPJ + k, 1)]

    def fire_gather(chunk, slot):
        return pltpu.async_copy(src(chunk), rows_v.at[slot, 0], gsem.at[slot])

    def fire_store(chunk, slot):
        return pltpu.async_copy(rows_v.at[slot], dst(chunk), ssem.at[slot])

    def wait_gather(chunk, slot):
        pltpu.make_async_copy(
            src(chunk), rows_v.at[slot, 0], gsem.at[slot]).wait()

    def wait_store(chunk, slot):
        pltpu.make_async_copy(
            rows_v.at[slot], dst(chunk), ssem.at[slot]).wait()

    for c in range(LA):
        fire_gather(c, c)
    for j in range(2):
        wait_gather(j, j)
        fire_store(j, j)
        fire_gather(j + LA, (j + LA) % NBUF)

    def group(g, carry):
        for k in range(NBUF):
            j = 2 + g * NBUF + k
            b = (2 + k) % NBUF
            s2 = (b + LA) % NBUF
            wait_gather(j, b)
            fire_store(j, b)
            wait_store(j - 2, s2)
            fire_gather(j + LA, s2)
        return carry

    G = (NCHUNK - 2 - LA) // NBUF
    lax.fori_loop(0, G, group, 0)

    for j in range(2 + G * NBUF, NCHUNK - LA):
        b = j % NBUF
        s2 = (b + LA) % NBUF
        wait_gather(j, b)
        fire_store(j, b)
        wait_store(j - 2, s2)
        fire_gather(j + LA, s2)
    for j in range(NCHUNK - LA, NCHUNK):
        b = j % NBUF
        wait_gather(j, b)
        fire_store(j, b)
        wait_store(j - 2, (b + LA) % NBUF)
    for j in range(NCHUNK - 2, NCHUNK):
        wait_store(j, j % NBUF)


@functools.partial(jax.jit, static_argnames=())
def _sc_gather(idx, table):
    kern = pl.kernel(
        _gather_body,
        out_type=jax.ShapeDtypeStruct((NBLK, CHUNK, D), jnp.float32),
        mesh=plsc.VectorSubcoreMesh(
            core_axis_name="c", subcore_axis_name="s",
            num_cores=NC, num_subcores=NS),
        scratch_types=[
            pltpu.VMEM((N_COLS, IPS), jnp.int32),
            pltpu.VMEM((NBUF, 1, CHUNK, D), jnp.float32),
            pltpu.SemaphoreType.DMA((NBUF,)),
            pltpu.SemaphoreType.DMA((NBUF,)),
        ],
        compiler_params=pltpu.CompilerParams(use_tc_tiling_on_sc=True),
    )
    return kern(idx, table)


def kernel(distance, FEATURE):
    idx = jnp.transpose(distance).astype(jnp.int32)
    out = _sc_gather(idx, FEATURE)
    return out.reshape(N_COLS, N_ROWS, D).transpose(1, 0, 2)
